# baseline (device time: 625236 ns/iter reference)
import jax
import jax.numpy as jnp
from jax import lax
from jax.experimental import pallas as pl
from jax.experimental.pallas import tpu as pltpu

N_DEV = 4
SQ = 256
D = 1024
HQ_LOC = 8
DH = 128
SKV = 4096
CHUNK = 1024
N_CH = SKV // CHUNK
SCALE = 0.08838834764831843
F32 = jnp.float32
BF16 = jnp.bfloat16

STEPS = ([(0, 0), (0, 1)]
         + [(r, j) for r in range(1, N_DEV) for j in range(N_CH)]
         + [(0, 2), (0, 3)])


def kernel(x, Wq, Wo, K_ext, V_ext):
    K2 = K_ext.reshape(N_DEV, SKV, 32 * DH)
    V2 = V_ext.reshape(N_DEV, SKV, 32 * DH)

    def body(x_ref, wq_ref, wo_ref, k_hbm, v_hbm, out_ref,
             ag_ref, part_ref, rs_ref, kbuf, vbuf, qbuf, obuf, qbuf0, obuf0,
             wqb, wob,
             ag_send, ag_recv, rs_send, rs_recv, ksem, vsem):
        i = lax.axis_index("i")
        right = lax.rem(i + 1, N_DEV)
        left = lax.rem(i + N_DEV - 1, N_DEV)
        col0 = i * (HQ_LOC * DH)

        def batch_of(r):
            return lax.rem(i + (N_DEV - r), N_DEV)

        barrier = pltpu.get_barrier_semaphore()
        for nbr in (left, right):
            pl.semaphore_signal(barrier, inc=1, device_id=(nbr,),
                                device_id_type=pl.DeviceIdType.MESH)
        pl.semaphore_wait(barrier, 2)

        def issue_kv(t):
            r, j = STEPS[t]
            b = batch_of(r)
            slot = t % 2
            ck = pltpu.make_async_copy(
                k_hbm.at[b, pl.ds(j * CHUNK, CHUNK), pl.ds(col0, HQ_LOC * DH)],
                kbuf.at[slot], ksem.at[slot])
            cv = pltpu.make_async_copy(
                v_hbm.at[b, pl.ds(j * CHUNK, CHUNK), pl.ds(col0, HQ_LOC * DH)],
                vbuf.at[slot], vsem.at[slot])
            ck.start()
            cv.start()
            return ck, cv

        kv_inflight = {0: issue_kv(0)}

        def attn_chunk(t, qsrc, o_acc, l_acc):
            if t + 1 < len(STEPS):
                kv_inflight[t + 1] = issue_kv(t + 1)
            ck, cv = kv_inflight.pop(t)
            ck.wait()
            cv.wait()
            slot = t % 2
            kb = kbuf[slot].astype(BF16)
            vb = vbuf[slot].astype(BF16)
            for h in range(HQ_LOC):
                cols = slice(h * DH, (h + 1) * DH)
                s = lax.dot_general(
                    qsrc[:, cols], kb[:, cols], (((1,), (1,)), ((), ())),
                    preferred_element_type=F32)
                p = jnp.exp(s)
                l_acc[h] = l_acc[h] + jnp.sum(p, axis=1, keepdims=True)
                o_acc[h] = o_acc[h] + jnp.dot(p.astype(BF16), vb[:, cols],
                                              preferred_element_type=F32)

        def finalize(o_acc, l_acc, osrc):
            for h in range(HQ_LOC):
                osrc[:, h * DH:(h + 1) * DH] = (o_acc[h] / l_acc[h]).astype(BF16)

        def fresh_acc():
            return ([jnp.zeros((SQ, DH), F32) for _ in range(HQ_LOC)],
                    [jnp.zeros((SQ, 1), F32) for _ in range(HQ_LOC)])

        def ag_rdma(h):
            src = x_ref.at[0] if h == 0 else ag_ref.at[h - 1]
            return pltpu.make_async_remote_copy(
                src_ref=src, dst_ref=ag_ref.at[h],
                send_sem=ag_send.at[h], recv_sem=ag_recv.at[h],
                device_id=(right,), device_id_type=pl.DeviceIdType.MESH)

        ag_inflight = ag_rdma(0)
        ag_inflight.start()

        def rs_rdma(s):
            return pltpu.make_async_remote_copy(
                src_ref=part_ref.at[s + 1], dst_ref=rs_ref.at[s],
                send_sem=rs_send.at[s], recv_sem=rs_recv.at[s],
                device_id=(right,), device_id_type=pl.DeviceIdType.MESH)

        qbuf0[...] = (jnp.dot(x_ref[0].astype(BF16), wqb[...],
                              preferred_element_type=F32) * SCALE).astype(BF16)
        o0, l0 = fresh_acc()
        attn_chunk(0, qbuf0, o0, l0)
        attn_chunk(1, qbuf0, o0, l0)

        rs_inflight = None
        for r in range(1, N_DEV):
            ag_inflight.wait()
            if r < N_DEV - 1:
                ag_inflight = ag_rdma(r)
                ag_inflight.start()
            qbuf[...] = (jnp.dot(ag_ref[r - 1].astype(BF16), wqb[...],
                                 preferred_element_type=F32)
                         * SCALE).astype(BF16)
            o_acc, l_acc = fresh_acc()
            for j in range(N_CH):
                attn_chunk(2 + (r - 1) * N_CH + j, qbuf, o_acc, l_acc)
            finalize(o_acc, l_acc, obuf)
            part_ref[r, :, :] = jnp.dot(obuf[...], wob[...],
                                        preferred_element_type=F32)
            if r >= 2:
                rs_inflight.wait()
                part_ref[r, :, :] = part_ref[r, :, :] + rs_ref[r - 2]
            rs_inflight = rs_rdma(r - 1)
            rs_inflight.start()

        attn_chunk(2 + 3 * N_CH, qbuf0, o0, l0)
        attn_chunk(3 + 3 * N_CH, qbuf0, o0, l0)
        finalize(o0, l0, obuf0)
        part0 = jnp.dot(obuf0[...], wob[...], preferred_element_type=F32)
        rs_inflight.wait()
        out_ref[0, :, :] = part0 + rs_ref[N_DEV - 2]

    def full_body(x_ref, wq_ref, wo_ref, k_hbm, v_hbm, out_ref,
                  ag_ref, part_ref, rs_ref, kbuf, vbuf, qbuf, obuf,
                  qbuf0, obuf0, wqb, wob,
                  ag_send, ag_recv, rs_send, rs_recv, ksem, vsem):
        wqb[...] = wq_ref[...].astype(BF16)
        wob[...] = wo_ref[...].astype(BF16)
        body(x_ref, wq_ref, wo_ref, k_hbm, v_hbm, out_ref,
             ag_ref, part_ref, rs_ref, kbuf, vbuf, qbuf, obuf,
             qbuf0, obuf0, wqb, wob,
             ag_send, ag_recv, rs_send, rs_recv, ksem, vsem)

    return pl.pallas_call(
        full_body,
        out_shape=jax.ShapeDtypeStruct((1, SQ, D), F32),
        in_specs=[
            pl.BlockSpec(memory_space=pltpu.VMEM),
            pl.BlockSpec(memory_space=pltpu.VMEM),
            pl.BlockSpec(memory_space=pltpu.VMEM),
            pl.BlockSpec(memory_space=pl.ANY),
            pl.BlockSpec(memory_space=pl.ANY),
        ],
        out_specs=pl.BlockSpec(memory_space=pltpu.VMEM),
        scratch_shapes=[
            pltpu.VMEM((N_DEV - 1, SQ, D), F32),
            pltpu.VMEM((N_DEV, SQ, D), F32),
            pltpu.VMEM((N_DEV - 1, SQ, D), F32),
            pltpu.VMEM((2, CHUNK, HQ_LOC * DH), F32),
            pltpu.VMEM((2, CHUNK, HQ_LOC * DH), F32),
            pltpu.VMEM((SQ, D), BF16),
            pltpu.VMEM((SQ, D), BF16),
            pltpu.VMEM((SQ, D), BF16),
            pltpu.VMEM((SQ, D), BF16),
            pltpu.VMEM((D, D), BF16),
            pltpu.VMEM((D, D), BF16),
            pltpu.SemaphoreType.DMA((N_DEV - 1,)),
            pltpu.SemaphoreType.DMA((N_DEV - 1,)),
            pltpu.SemaphoreType.DMA((N_DEV - 1,)),
            pltpu.SemaphoreType.DMA((N_DEV - 1,)),
            pltpu.SemaphoreType.DMA((2,)),
            pltpu.SemaphoreType.DMA((2,)),
        ],
        compiler_params=pltpu.CompilerParams(
            collective_id=0,
            vmem_limit_bytes=63 * 1024 * 1024,
        ),
    )(x, Wq, Wo, K2, V2)


# device time: 87582 ns/iter; 7.1389x vs baseline; 7.1389x over previous
import jax
import jax.numpy as jnp
from jax import lax
from jax.experimental import pallas as pl
from jax.experimental.pallas import tpu as pltpu

N_DEV = 4
SQ = 256
D = 1024
HQ_LOC = 8
DH = 128
SKV = 4096
SCALE = 0.08838834764831843
F32 = jnp.float32
BF16 = jnp.bfloat16
NSLOT = 3

STEPS = ([(0, h) for h in range(4)]
         + [(r, h) for r in range(1, N_DEV) for h in range(HQ_LOC)]
         + [(0, h) for h in range(4, HQ_LOC)])


def kernel(x, Wq, Wo, K_ext, V_ext):
    def body(x_ref, wq_ref, wo_ref, k_hbm, v_hbm, out_ref,
             ag_ref, part_ref, rs_ref, kbuf, vbuf, qbuf, obuf, qbuf0, obuf0,
             wqb, wob,
             ag_send, ag_recv, rs_send, rs_recv, ksem, vsem):
        i = lax.axis_index("i")
        right = lax.rem(i + 1, N_DEV)
        left = lax.rem(i + N_DEV - 1, N_DEV)
        h0 = i * HQ_LOC

        def batch_of(r):
            return lax.rem(i + (N_DEV - r), N_DEV)

        barrier = pltpu.get_barrier_semaphore()
        for nbr in (left, right):
            pl.semaphore_signal(barrier, inc=1, device_id=(nbr,),
                                device_id_type=pl.DeviceIdType.MESH)
        pl.semaphore_wait(barrier, 2)

        def issue_kv(t):
            r, h = STEPS[t]
            b = batch_of(r)
            slot = t % NSLOT
            ck = pltpu.make_async_copy(
                k_hbm.at[b, :, h0 + h, :], kbuf.at[slot], ksem.at[slot])
            cv = pltpu.make_async_copy(
                v_hbm.at[b, :, h0 + h, :], vbuf.at[slot], vsem.at[slot])
            ck.start()
            cv.start()
            return ck, cv

        kv_inflight = {t: issue_kv(t) for t in range(NSLOT - 1)}

        def attn_step(t, qsrc, osrc):
            r, h = STEPS[t]
            if t + NSLOT - 1 < len(STEPS):
                kv_inflight[t + NSLOT - 1] = issue_kv(t + NSLOT - 1)
            ck, cv = kv_inflight.pop(t)
            ck.wait()
            cv.wait()
            slot = t % NSLOT
            qh = qsrc[:, h * DH:(h + 1) * DH]
            s = lax.dot_general(
                qh, kbuf[slot].astype(BF16), (((1,), (1,)), ((), ())),
                preferred_element_type=F32)
            p = jnp.exp(s)
            l = jnp.sum(p, axis=1, keepdims=True)
            o = jnp.dot(p.astype(BF16), vbuf[slot].astype(BF16),
                        preferred_element_type=F32) / l
            osrc[:, h * DH:(h + 1) * DH] = o.astype(BF16)

        def ag_rdma(h):
            src = x_ref.at[0] if h == 0 else ag_ref.at[h - 1]
            return pltpu.make_async_remote_copy(
                src_ref=src, dst_ref=ag_ref.at[h],
                send_sem=ag_send.at[h], recv_sem=ag_recv.at[h],
                device_id=(right,), device_id_type=pl.DeviceIdType.MESH)

        ag_inflight = ag_rdma(0)
        ag_inflight.start()

        def rs_rdma(s):
            return pltpu.make_async_remote_copy(
                src_ref=part_ref.at[s + 1], dst_ref=rs_ref.at[s],
                send_sem=rs_send.at[s], recv_sem=rs_recv.at[s],
                device_id=(right,), device_id_type=pl.DeviceIdType.MESH)

        qbuf0[...] = (jnp.dot(x_ref[0].astype(BF16), wqb[...],
                              preferred_element_type=F32) * SCALE).astype(BF16)
        for t in range(4):
            attn_step(t, qbuf0, obuf0)

        rs_inflight = None
        for r in range(1, N_DEV):
            ag_inflight.wait()
            if r < N_DEV - 1:
                ag_inflight = ag_rdma(r)
                ag_inflight.start()
            qbuf[...] = (jnp.dot(ag_ref[r - 1].astype(BF16), wqb[...],
                                 preferred_element_type=F32)
                         * SCALE).astype(BF16)
            for h in range(HQ_LOC):
                attn_step(4 + (r - 1) * HQ_LOC + h, qbuf, obuf)
            part_ref[r, :, :] = jnp.dot(obuf[...], wob[...],
                                        preferred_element_type=F32)
            if r >= 2:
                rs_inflight.wait()
                part_ref[r, :, :] = part_ref[r, :, :] + rs_ref[r - 2]
            rs_inflight = rs_rdma(r - 1)
            rs_inflight.start()

        for t in range(4 + 3 * HQ_LOC, len(STEPS)):
            attn_step(t, qbuf0, obuf0)
        part0 = jnp.dot(obuf0[...], wob[...], preferred_element_type=F32)
        rs_inflight.wait()
        out_ref[0, :, :] = part0 + rs_ref[N_DEV - 2]

    def full_body(x_ref, wq_ref, wo_ref, k_hbm, v_hbm, out_ref,
                  ag_ref, part_ref, rs_ref, kbuf, vbuf, qbuf, obuf,
                  qbuf0, obuf0, wqb, wob,
                  ag_send, ag_recv, rs_send, rs_recv, ksem, vsem):
        wqb[...] = wq_ref[...].astype(BF16)
        wob[...] = wo_ref[...].astype(BF16)
        body(x_ref, wq_ref, wo_ref, k_hbm, v_hbm, out_ref,
             ag_ref, part_ref, rs_ref, kbuf, vbuf, qbuf, obuf,
             qbuf0, obuf0, wqb, wob,
             ag_send, ag_recv, rs_send, rs_recv, ksem, vsem)

    return pl.pallas_call(
        full_body,
        out_shape=jax.ShapeDtypeStruct((1, SQ, D), F32),
        in_specs=[
            pl.BlockSpec(memory_space=pltpu.VMEM),
            pl.BlockSpec(memory_space=pltpu.VMEM),
            pl.BlockSpec(memory_space=pltpu.VMEM),
            pl.BlockSpec(memory_space=pl.ANY),
            pl.BlockSpec(memory_space=pl.ANY),
        ],
        out_specs=pl.BlockSpec(memory_space=pltpu.VMEM),
        scratch_shapes=[
            pltpu.VMEM((N_DEV - 1, SQ, D), F32),
            pltpu.VMEM((N_DEV, SQ, D), F32),
            pltpu.VMEM((N_DEV - 1, SQ, D), F32),
            pltpu.VMEM((NSLOT, SKV, DH), F32),
            pltpu.VMEM((NSLOT, SKV, DH), F32),
            pltpu.VMEM((SQ, D), BF16),
            pltpu.VMEM((SQ, D), BF16),
            pltpu.VMEM((SQ, D), BF16),
            pltpu.VMEM((SQ, D), BF16),
            pltpu.VMEM((D, D), BF16),
            pltpu.VMEM((D, D), BF16),
            pltpu.SemaphoreType.DMA((N_DEV - 1,)),
            pltpu.SemaphoreType.DMA((N_DEV - 1,)),
            pltpu.SemaphoreType.DMA((N_DEV - 1,)),
            pltpu.SemaphoreType.DMA((N_DEV - 1,)),
            pltpu.SemaphoreType.DMA((NSLOT,)),
            pltpu.SemaphoreType.DMA((NSLOT,)),
        ],
        compiler_params=pltpu.CompilerParams(
            collective_id=0,
            vmem_limit_bytes=63 * 1024 * 1024,
        ),
    )(x, Wq, Wo, K_ext, V_ext)
